# baseline (speedup 1.0000x reference)
import functools

import jax
import jax.numpy as jnp
from jax import lax
from jax.experimental import pallas as pl
from jax.experimental.pallas import tpu as pltpu
from jax.experimental.pallas import tpu_sc as plsc

D_MODEL = 64
SCALE = 8.0
LANES = 16
GB = 4
SUB = (32, 18)


@functools.partial(jax.jit, static_argnames=("n_workers",))
def _embed_sc(x, tpad, *, n_workers):
    batch, seq = x.shape
    bpw = batch // n_workers
    info = plsc.get_sparse_core_info()
    nc, ns = info.num_cores, info.num_subcores
    assert nc * ns == n_workers
    mesh = plsc.VectorSubcoreMesh(core_axis_name="c", subcore_axis_name="s")

    @functools.partial(
        pl.kernel,
        mesh=mesh,
        out_type=jax.ShapeDtypeStruct((batch, seq, D_MODEL), jnp.float32),
        scratch_types=[
            pltpu.VMEM((bpw, seq), jnp.int32),
            pltpu.VMEM((32, 1, 128), jnp.float32),
            pltpu.VMEM((GB, seq, D_MODEL), jnp.float32),
            pltpu.SemaphoreType.DMA,
        ],
    )
    def body(table_hbm, x_hbm, out_hbm, idx_v, gbuf, stage, gsem):
        wid = lax.axis_index("s") * nc + lax.axis_index("c")
        b0 = wid * bpw
        pltpu.sync_copy(x_hbm.at[pl.ds(b0, bpw)], idx_v)

        for g in range(bpw // GB):
            for bg in range(GB):
                bi = g * GB + bg
                t0 = 0
                for n in SUB:
                    idx_sl = idx_v.at[bi, pl.ds(t0, n)]
                    pltpu.async_copy(
                        table_hbm.at[idx_sl], gbuf.at[pl.ds(0, n)], gsem
                    ).wait()

                    def extract(k, _, bg=bg, t0=t0):
                        for j in range(D_MODEL // LANES):
                            sl = pl.ds(j * LANES, LANES)
                            stage[bg, t0 + k, sl] = gbuf[k, 0, sl] * SCALE
                        return 0

                    lax.fori_loop(0, n, extract, 0)
                    t0 += n
            pltpu.sync_copy(stage, out_hbm.at[pl.ds(b0 + g * GB, GB)])

    return body(tpad, x)


def kernel(x, table):
    n_workers = 32
    tpad = jnp.pad(table.reshape(table.shape[0], 1, D_MODEL),
                   ((0, 0), (0, 0), (0, 128 - D_MODEL)))
    return _embed_sc(x, tpad, n_workers=n_workers)


# speedup vs baseline: 1.4013x; 1.4013x over previous
import functools

import jax
import jax.numpy as jnp
from jax import lax
from jax.experimental import pallas as pl
from jax.experimental.pallas import tpu as pltpu
from jax.experimental.pallas import tpu_sc as plsc

D_MODEL = 64
SCALE = 8.0
CHUNK = 128
GROUP = 5 * CHUNK
LANES = 16
ROW_UNROLL = 4


@functools.partial(jax.jit, static_argnames=("n_groups", "n_workers"))
def _embed_sc(x1d, table, *, n_groups, n_workers):
    b_total = x1d.shape[0]
    bpw = n_groups * GROUP
    info = plsc.get_sparse_core_info()
    nc, ns = info.num_cores, info.num_subcores
    assert nc * ns == n_workers
    mesh = plsc.VectorSubcoreMesh(core_axis_name="c", subcore_axis_name="s")

    @functools.partial(
        pl.kernel,
        mesh=mesh,
        compiler_params=pltpu.CompilerParams(use_tc_tiling_on_sc=False),
        out_type=jax.ShapeDtypeStruct((b_total, D_MODEL), jnp.float32),
        scratch_types=[
            pltpu.VMEM((bpw,), jnp.int32),
            pltpu.VMEM((GROUP, D_MODEL), jnp.float32),
            pltpu.VMEM((GROUP, D_MODEL), jnp.float32),
            pltpu.SemaphoreType.DMA,
            pltpu.SemaphoreType.DMA,
        ],
    )
    def body(table_hbm, idx_hbm, out_hbm, idx_v, rows_a, rows_b, sem_a, sem_b):
        wid = lax.axis_index("s") * nc + lax.axis_index("c")
        pltpu.sync_copy(idx_hbm.at[pl.ds(wid * bpw, bpw)], idx_v)
        out_base = wid * bpw
        bufs = (rows_a, rows_b)
        sems = (sem_a, sem_b)

        def fire(g, b):
            descs = []
            for c in range(GROUP // CHUNK):
                idx_sl = idx_v.at[pl.ds(g * GROUP + c * CHUNK, CHUNK)]
                dst = bufs[b].at[pl.ds(c * CHUNK, CHUNK)]
                descs.append(pltpu.async_copy(table_hbm.at[idx_sl], dst, sems[b]))
            return descs

        def scale(b):
            rows = bufs[b]

            def scale_rows(i, _):
                r0 = i * ROW_UNROLL
                for dr in range(ROW_UNROLL):
                    for j in range(D_MODEL // LANES):
                        sl = pl.ds(j * LANES, LANES)
                        rows[r0 + dr, sl] = rows[r0 + dr, sl] * SCALE
                return 0

            lax.fori_loop(0, GROUP // ROW_UNROLL, scale_rows, 0)

        in_flight = {0: fire(0, 0)}
        for g in range(n_groups):
            b = g & 1
            if g + 1 < n_groups:
                in_flight[g + 1] = fire(g + 1, 1 - b)
            for d in in_flight.pop(g):
                d.wait()
            scale(b)
            pltpu.sync_copy(bufs[b], out_hbm.at[pl.ds(out_base + g * GROUP, GROUP)])

    return body(table, x1d)


def kernel(x, table):
    b = x.size
    n_workers = 32
    assert b % (n_workers * GROUP) == 0
    n_groups = b // (n_workers * GROUP)
    out = _embed_sc(x.reshape(-1), table, n_groups=n_groups, n_workers=n_workers)
    return out.reshape(x.shape + (D_MODEL,))
